# Initial kernel scaffold; baseline (speedup 1.0000x reference)
import jax
import jax.numpy as jnp
from jax.experimental import pallas as pl


def kernel(idx, table):
    raise NotImplementedError("write your pallas kernel here")


# speedup vs baseline: 19.4256x; 19.4256x over previous
import functools

import jax
import jax.numpy as jnp
from jax import lax
from jax.experimental import pallas as pl
from jax.experimental.pallas import tpu as pltpu
from jax.experimental.pallas import tpu_sc as plsc

CHUNK = 128


@functools.lru_cache(maxsize=None)
def _build(B, V, D):
    info = plsc.get_sparse_core_info()
    NC, NS = info.num_cores, info.num_subcores
    NW = NC * NS
    assert B % (NW * CHUNK) == 0
    b_per_w = B // NW
    n_chunks = b_per_w // CHUNK
    mesh = plsc.VectorSubcoreMesh(core_axis_name="c", subcore_axis_name="s")

    @functools.partial(
        pl.kernel,
        mesh=mesh,
        out_type=jax.ShapeDtypeStruct((B, D), jnp.float32),
        compiler_params=pltpu.CompilerParams(use_tc_tiling_on_sc=False),
        scratch_types=[
            pltpu.VMEM((n_chunks, CHUNK), jnp.int32),
            pltpu.VMEM((CHUNK, D), jnp.float32),
            pltpu.SemaphoreType.DMA,
        ],
    )
    def k(idx_hbm, table_hbm, out_hbm, idx_v, rows_v, sem):
        wid = lax.axis_index("s") * NC + lax.axis_index("c")
        pltpu.sync_copy(idx_hbm.at[wid], idx_v)

        def body(j, carry):
            pltpu.async_copy(table_hbm.at[idx_v.at[j]], rows_v, sem).wait()
            pltpu.sync_copy(
                rows_v, out_hbm.at[pl.ds(wid * b_per_w + j * CHUNK, CHUNK)]
            )
            return carry

        lax.fori_loop(0, n_chunks, body, 0)

    return k


def kernel(idx, table):
    bsz, slen = idx.shape
    V, D = table.shape
    B = bsz * slen
    info = plsc.get_sparse_core_info()
    nw = info.num_cores * info.num_subcores
    idx3d = idx.reshape(nw, B // (nw * CHUNK), CHUNK)
    out = _build(B, V, D)(idx3d, table)
    return out.reshape(bsz, slen, D)


# speedup vs baseline: 22.1788x; 1.1417x over previous
import functools

import jax
import jax.numpy as jnp
from jax import lax
from jax.experimental import pallas as pl
from jax.experimental.pallas import tpu as pltpu
from jax.experimental.pallas import tpu_sc as plsc

CHUNK = 128
NBUF = 10


@functools.lru_cache(maxsize=None)
def _build(B, V, D):
    info = plsc.get_sparse_core_info()
    NC, NS = info.num_cores, info.num_subcores
    NW = NC * NS
    assert B % (NW * CHUNK) == 0
    b_per_w = B // NW
    n_chunks = b_per_w // CHUNK
    assert n_chunks % NBUF == 0
    n_groups = n_chunks // NBUF
    mesh = plsc.VectorSubcoreMesh(core_axis_name="c", subcore_axis_name="s")

    @functools.partial(
        pl.kernel,
        mesh=mesh,
        out_type=jax.ShapeDtypeStruct((B, D), jnp.float32),
        compiler_params=pltpu.CompilerParams(use_tc_tiling_on_sc=False),
        scratch_types=[
            pltpu.VMEM((n_chunks, CHUNK), jnp.int32),
            pltpu.VMEM((NBUF, CHUNK, D), jnp.float32),
            pltpu.SemaphoreType.DMA((NBUF,)),
            pltpu.SemaphoreType.DMA((NBUF,)),
        ],
    )
    def k(idx_hbm, table_hbm, out_hbm, idx_v, bufs, sem_g, sem_w):
        wid = lax.axis_index("s") * NC + lax.axis_index("c")
        base = wid * b_per_w
        pltpu.sync_copy(idx_hbm.at[wid], idx_v)

        for b in range(NBUF):
            pltpu.async_copy(table_hbm.at[idx_v.at[b]], bufs.at[b], sem_g.at[b])

        def group(g, carry):
            for b in range(NBUF):
                j = g * NBUF + b
                pltpu.make_async_copy(
                    table_hbm.at[idx_v.at[j]], bufs.at[b], sem_g.at[b]
                ).wait()
                pltpu.async_copy(
                    bufs.at[b],
                    out_hbm.at[pl.ds(base + j * CHUNK, CHUNK)],
                    sem_w.at[b],
                )

                @pl.when(g + 1 < n_groups)
                def _():
                    pltpu.make_async_copy(
                        bufs.at[b],
                        out_hbm.at[pl.ds(base + j * CHUNK, CHUNK)],
                        sem_w.at[b],
                    ).wait()
                    pltpu.async_copy(
                        table_hbm.at[idx_v.at[j + NBUF]], bufs.at[b], sem_g.at[b]
                    )

            return carry

        lax.fori_loop(0, n_groups, group, 0)

        for b in range(NBUF):
            j = (n_groups - 1) * NBUF + b
            pltpu.make_async_copy(
                bufs.at[b],
                out_hbm.at[pl.ds(base + j * CHUNK, CHUNK)],
                sem_w.at[b],
            ).wait()

    return k


def kernel(idx, table):
    bsz, slen = idx.shape
    V, D = table.shape
    B = bsz * slen
    info = plsc.get_sparse_core_info()
    nw = info.num_cores * info.num_subcores
    idx3d = idx.reshape(nw, B // (nw * CHUNK), CHUNK)
    out = _build(B, V, D)(idx3d, table)
    return out.reshape(bsz, slen, D)
